# Initial kernel scaffold; baseline (speedup 1.0000x reference)
import jax
import jax.numpy as jnp
from jax.experimental import pallas as pl


def kernel(x, edge_index, Wl1, Wr1, b1, Wl2, Wr2, b2):
    raise NotImplementedError("write your pallas kernel here")


# speedup vs baseline: 5.6504x; 5.6504x over previous
import functools

import jax
import jax.numpy as jnp
from jax import lax
from jax.experimental import pallas as pl
from jax.experimental.pallas import tpu as pltpu
from jax.experimental.pallas import tpu_sc as plsc

N_NODES = 10000
N_PAD = 10240
FEAT = 128
N_EDGES = 320000
NC = 2
NS = 16
NW = NC * NS
EPW = N_EDGES // NW
CHUNK = 80
NCHUNKS = EPW // CHUNK
RPT = N_PAD // NS
ZR = 128

def _fill(ref, val, nrows, ncols):
    val16 = jnp.full((16,), val, jnp.float32)
    if ncols is None:
        def body(i, _):
            ref[pl.ds(i * 16, 16)] = val16
            return 0
        lax.fori_loop(0, nrows // 16, body, 0)
    else:
        def row(i, _):
            def col(j, _):
                ref[i, pl.ds(j * 16, 16)] = val16
                return 0
            return lax.fori_loop(0, ncols // 16, col, 0)
        lax.fori_loop(0, nrows, row, 0)


def _make_agg(with_cnt: bool):
    mesh = plsc.VectorSubcoreMesh(core_axis_name="c", subcore_axis_name="s")
    out_type = [jax.ShapeDtypeStruct((NC, N_PAD, FEAT), jnp.float32)]
    scratch = [
        pltpu.VMEM((CHUNK,), jnp.int32),
        pltpu.VMEM((CHUNK,), jnp.int32),
        pltpu.VMEM((CHUNK, FEAT), jnp.float32),
        pltpu.VMEM((ZR, FEAT), jnp.float32),
        pltpu.VMEM_SHARED((N_PAD, FEAT), jnp.float32),
        pltpu.SemaphoreType.DMA,
    ]
    if with_cnt:
        out_type.append(jax.ShapeDtypeStruct((NC, N_PAD), jnp.float32))
        scratch += [
            pltpu.VMEM((CHUNK,), jnp.float32),
            pltpu.VMEM((RPT,), jnp.float32),
            pltpu.VMEM_SHARED((N_PAD,), jnp.float32),
        ]

    def body(feat, src, dst, *rest):
        if with_cnt:
            (out_sum, out_cnt, sidx, didx, rows, zbuf, ssum, sem,
             ones, zc, scnt) = rest
        else:
            out_sum, sidx, didx, rows, zbuf, ssum, sem = rest
        cid = lax.axis_index("c")
        sid = lax.axis_index("s")
        wid = cid * NS + sid
        rb = sid * RPT

        _fill(zbuf, 0.0, ZR, FEAT)
        for k in range(RPT // ZR):
            pltpu.sync_copy(zbuf, ssum.at[pl.ds(rb + k * ZR, ZR), :])
        if with_cnt:
            _fill(ones, 1.0, CHUNK, None)
            _fill(zc, 0.0, RPT, None)
            pltpu.sync_copy(zc, scnt.at[pl.ds(rb, RPT)])
        plsc.subcore_barrier()

        eb0 = wid * EPW

        def chunk(i, _):
            eb = eb0 + i * CHUNK
            pltpu.sync_copy(src.at[pl.ds(eb, CHUNK)], sidx)
            pltpu.sync_copy(dst.at[pl.ds(eb, CHUNK)], didx)
            pltpu.async_copy(feat.at[sidx], rows, sem).wait()
            pltpu.sync_copy(rows, ssum.at[didx], add=True)
            if with_cnt:
                pltpu.sync_copy(ones, scnt.at[didx], add=True)
            return 0

        lax.fori_loop(0, NCHUNKS, chunk, 0)
        plsc.subcore_barrier()

        pltpu.sync_copy(ssum.at[pl.ds(rb, RPT), :],
                        out_sum.at[cid, pl.ds(rb, RPT), :])
        if with_cnt:
            pltpu.sync_copy(scnt.at[pl.ds(rb, RPT)],
                            out_cnt.at[cid, pl.ds(rb, RPT)])

    return pl.kernel(body, out_type=out_type, mesh=mesh,
                     scratch_types=scratch)


_agg_cnt = _make_agg(True)
_agg = _make_agg(False)

BR = 2048


def _dense_body(relu):
    def body(sp_ref, cp_ref, x_ref, wl_ref, wr_ref, b_ref, o_ref):
        c = jnp.clip(cp_ref[0] + cp_ref[1], 1.0, None)
        mean = (sp_ref[0] + sp_ref[1]) / c[:, None]
        acc = jnp.dot(mean, wl_ref[...], preferred_element_type=jnp.float32)
        acc = acc + jnp.dot(x_ref[...], wr_ref[...],
                            preferred_element_type=jnp.float32)
        acc = acc + b_ref[...]
        if relu:
            acc = jnp.maximum(acc, 0.0)
        o_ref[...] = acc
    return body


def _dense_layer(sp, cp, x, wl, wr, b, relu):
    return pl.pallas_call(
        _dense_body(relu),
        grid=(N_PAD // BR,),
        in_specs=[
            pl.BlockSpec((NC, BR, FEAT), lambda i: (0, i, 0)),
            pl.BlockSpec((NC, BR), lambda i: (0, i)),
            pl.BlockSpec((BR, FEAT), lambda i: (i, 0)),
            pl.BlockSpec((FEAT, FEAT), lambda i: (0, 0)),
            pl.BlockSpec((FEAT, FEAT), lambda i: (0, 0)),
            pl.BlockSpec((1, FEAT), lambda i: (0, 0)),
        ],
        out_specs=pl.BlockSpec((BR, FEAT), lambda i: (i, 0)),
        out_shape=jax.ShapeDtypeStruct((N_PAD, FEAT), jnp.float32),
    )(sp, cp, x, wl, wr, b)


def kernel(x, edge_index, Wl1, Wr1, b1, Wl2, Wr2, b2):
    src = edge_index[0].astype(jnp.int32)
    dst = edge_index[1].astype(jnp.int32)
    x_pad = jnp.pad(x, ((0, N_PAD - N_NODES), (0, 0)))
    sp1, cp = _agg_cnt(x_pad, src, dst)
    h = _dense_layer(sp1, cp, x_pad, Wl1, Wr1, b1.reshape(1, FEAT), True)
    sp2, = _agg(h, src, dst)
    out = _dense_layer(sp2, cp, h, Wl2, Wr2, b2.reshape(1, FEAT), False)
    return out[:N_NODES]
